# baseline (speedup 1.0000x reference)
import functools

import jax
import jax.numpy as jnp
from jax import lax
from jax.experimental import pallas as pl
from jax.experimental.pallas import tpu as pltpu
from jax.experimental.pallas import tpu_sc as plsc

_NC = 2
_NS = 16
_LANES = 16


def _build(B, L, V, D):
    T = B * L
    NW = _NC * _NS
    per_w = T // NW
    C = L
    n_chunks = per_w // C

    mesh = plsc.VectorSubcoreMesh(core_axis_name="c", subcore_axis_name="s")

    @functools.partial(
        pl.kernel,
        out_type=jax.ShapeDtypeStruct((T, D), jnp.float32),
        mesh=mesh,
        scratch_types=[
            pltpu.VMEM((C,), jnp.int32),
            pltpu.VMEM((C, D), jnp.float32),
            pltpu.VMEM((L, D), jnp.float32),
            pltpu.SemaphoreType.DMA,
        ],
        compiler_params=pltpu.CompilerParams(use_tc_tiling_on_sc=False),
    )
    def run(tok_hbm, tab_hbm, pos_hbm, out_hbm, idx_v, rows_v, pos_v, sem):
        wid = lax.axis_index("s") * _NC + lax.axis_index("c")
        base = wid * per_w
        pltpu.sync_copy(pos_hbm, pos_v)

        def chunk_body(g, carry):
            off = base + g * C
            pltpu.sync_copy(tok_hbm.at[pl.ds(off, C)], idx_v)
            pltpu.async_copy(tab_hbm.at[idx_v], rows_v, sem).wait()

            def add_row(r, c2):
                for j in range(D // _LANES):
                    sl = pl.ds(j * _LANES, _LANES)
                    plsc.addupdate(rows_v.at[r, sl], pos_v[r, sl])
                return c2

            lax.fori_loop(0, C, add_row, 0, unroll=2)
            pltpu.sync_copy(rows_v, out_hbm.at[pl.ds(off, C)])
            return carry

        lax.fori_loop(0, n_chunks, chunk_body, 0)

    return run


def kernel(tokens, token_table, pos_emb):
    B, L = tokens.shape
    V, D = token_table.shape
    run = _build(B, L, V, D)
    out = run(tokens.reshape(B * L), token_table, pos_emb)
    return out.reshape(B, L, D)


# speedup vs baseline: 1.1568x; 1.1568x over previous
import functools

import jax
import jax.numpy as jnp
from jax import lax
from jax.experimental import pallas as pl
from jax.experimental.pallas import tpu as pltpu
from jax.experimental.pallas import tpu_sc as plsc

_NC = 2
_NS = 16
_LANES = 16
_NBUF = 4
_LOOK = 2


def _build(B, L, V, D):
    T = B * L
    NW = _NC * _NS
    per_w = T // NW
    C = L
    n_chunks = per_w // C

    mesh = plsc.VectorSubcoreMesh(core_axis_name="c", subcore_axis_name="s")

    scratch = (
        [pltpu.VMEM((C,), jnp.int32) for _ in range(_NBUF)]
        + [pltpu.VMEM((C, D), jnp.float32) for _ in range(_NBUF)]
        + [pltpu.VMEM((L, D), jnp.float32)]
        + [pltpu.SemaphoreType.DMA for _ in range(2 * _NBUF)]
    )

    @functools.partial(
        pl.kernel,
        out_type=jax.ShapeDtypeStruct((T, D), jnp.float32),
        mesh=mesh,
        scratch_types=scratch,
        compiler_params=pltpu.CompilerParams(use_tc_tiling_on_sc=False),
    )
    def run(tok_hbm, tab_hbm, pos_hbm, out_hbm, *scr):
        idx = scr[0:_NBUF]
        rows = scr[_NBUF:2 * _NBUF]
        pos_v = scr[2 * _NBUF]
        gsem = scr[2 * _NBUF + 1: 2 * _NBUF + 1 + _NBUF]
        osem = scr[2 * _NBUF + 1 + _NBUF:]

        wid = lax.axis_index("s") * _NC + lax.axis_index("c")
        base = wid * per_w
        pltpu.sync_copy(pos_hbm, pos_v)

        for g in range(_LOOK):
            b = g % _NBUF
            pltpu.sync_copy(tok_hbm.at[pl.ds(base + g * C, C)], idx[b])
            pltpu.async_copy(tab_hbm.at[idx[b]], rows[b], gsem[b])

        def visit(g, bo):
            bl = (bo + _LOOK) % _NBUF

            @pl.when(g + _LOOK < n_chunks)
            def _launch():
                pltpu.sync_copy(
                    tok_hbm.at[pl.ds(base + (g + _LOOK) * C, C)], idx[bl])

                @pl.when(g + _LOOK - _NBUF >= 0)
                def _drain():
                    pltpu.make_async_copy(
                        rows[bl], out_hbm.at[pl.ds(base, C)], osem[bl]).wait()

                pltpu.async_copy(tab_hbm.at[idx[bl]], rows[bl], gsem[bl])

            pltpu.make_async_copy(
                tab_hbm.at[idx[bo]], rows[bo], gsem[bo]).wait()

            def add_row(r, c2):
                for j in range(D // _LANES):
                    sl = pl.ds(j * _LANES, _LANES)
                    plsc.addupdate(rows[bo].at[r, sl], pos_v[r, sl])
                return c2

            lax.fori_loop(0, C, add_row, 0, unroll=4)
            pltpu.async_copy(
                rows[bo], out_hbm.at[pl.ds(base + g * C, C)], osem[bo])

        def outer(m, carry):
            for bo in range(_NBUF):
                visit(m * _NBUF + bo, bo)
            return carry

        lax.fori_loop(0, n_chunks // _NBUF, outer, 0)

        for bo in range(_NBUF):
            pltpu.make_async_copy(
                rows[bo], out_hbm.at[pl.ds(base, C)], osem[bo]).wait()

    return run


def kernel(tokens, token_table, pos_emb):
    B, L = tokens.shape
    V, D = token_table.shape
    run = _build(B, L, V, D)
    out = run(tokens.reshape(B * L), token_table, pos_emb)
    return out.reshape(B, L, D)
